# baseline (speedup 1.0000x reference)
import functools

import jax
import jax.numpy as jnp
from jax import lax
from jax.experimental import pallas as pl
from jax.experimental.pallas import tpu as pltpu
from jax.experimental.pallas import tpu_sc as plsc

B = 4096
F = 26
V = 100000
D = 16
BF = B * F

_info = plsc.get_sparse_core_info()
NC = _info.num_cores
NS = _info.num_subcores
L = _info.num_lanes
NW = NC * NS
BPW = BF // NW
NGRP = BPW // L
PERIOD = 208 // L
CHUNK = 128
NCHUNK = BPW // CHUNK
GPC = CHUNK // L


def _body(idx_hbm, tab_hbm, out_hbm, idx_v, off_v, rows_v, gsem):
    wid = lax.axis_index("s") * NC + lax.axis_index("c")
    base = wid * BPW
    pltpu.sync_copy(idx_hbm.at[pl.ds(base, BPW)], idx_v)

    lane = lax.iota(jnp.int32, L)
    for g in range(PERIOD):
        off_v[pl.ds(g * L, L)] = ((g * L + lane) % F) * V

    for j in range(NCHUNK):
        for g in range(GPC):
            grp = j * GPC + g
            p = grp * L
            q = (grp % PERIOD) * L
            idx_v[pl.ds(p, L)] = idx_v[pl.ds(p, L)] + off_v[pl.ds(q, L)]

    handles = []
    for j in range(NCHUNK):
        handles.append(
            pltpu.async_copy(
                tab_hbm.at[idx_v.at[pl.ds(j * CHUNK, CHUNK)]],
                rows_v.at[pl.ds(j * CHUNK, CHUNK)],
                gsem,
            )
        )
    for h in handles:
        h.wait()

    pltpu.sync_copy(rows_v, out_hbm.at[pl.ds(base, BPW)])


_mesh = plsc.VectorSubcoreMesh(core_axis_name="c", subcore_axis_name="s")

_gather = pl.kernel(
    _body,
    out_type=jax.ShapeDtypeStruct((BF, D), jnp.float32),
    mesh=_mesh,
    scratch_types=[
        pltpu.VMEM((BPW,), jnp.int32),
        pltpu.VMEM((PERIOD * L,), jnp.int32),
        pltpu.VMEM((BPW, D), jnp.float32),
        pltpu.SemaphoreType.DMA,
    ],
    compiler_params=pltpu.CompilerParams(use_tc_tiling_on_sc=False),
)


@jax.jit
def kernel(indices, tables):
    idx_flat = indices.reshape(BF)
    tab_flat = tables.reshape(F * V, D)
    out = _gather(idx_flat, tab_flat)
    return out.reshape(B, F, D)


# speedup vs baseline: 1.0291x; 1.0291x over previous
import jax
import jax.numpy as jnp
from jax import lax
from jax.experimental import pallas as pl
from jax.experimental.pallas import tpu as pltpu
from jax.experimental.pallas import tpu_sc as plsc

B = 4096
F = 26
V = 100000
D = 16
BF = B * F
FV8 = F * V // 8
BF8 = BF // 8

_info = plsc.get_sparse_core_info()
NC = _info.num_cores
NS = _info.num_subcores
L = _info.num_lanes
NW = NC * NS
BPW = BF // NW
OPW = BPW // 8
CHUNK = 128
NCHUNK = BPW // CHUNK
GPC = CHUNK // L

def _body(idx_hbm, tab_hbm, out_hbm, idx_v, g_v, sub_v, out_v, buf0, buf1,
          sem0, sem1, osem):
    _LANE = lax.iota(jnp.int32, L)
    wid = lax.axis_index("s") * NC + lax.axis_index("c")
    base = wid * BPW
    pltpu.sync_copy(idx_hbm.at[pl.ds(base, BPW)], idx_v)

    bufs = (buf0, buf1)
    sems = (sem0, sem1)

    def prep_chunk(j):
        for g in range(GPC):
            p0 = j * CHUNK + g * L
            p = p0 + _LANE
            r = idx_v[pl.ds(p0, L)] + (p % F) * V
            g_v[pl.ds(p0, L)] = lax.shift_right_logical(r, 3)
            sub_v[pl.ds(p0, L)] = lax.shift_left((r & 7), 4)

    def fire(j, b):
        return pltpu.async_copy(
            tab_hbm.at[g_v.at[pl.ds(j * CHUNK, CHUNK)]], bufs[b], sems[b])

    def extract(j, b):
        for t in range(GPC):
            rows = t * L + _LANE
            sub16 = sub_v[pl.ds(j * CHUNK + t * L, L)]
            orow = (j * (CHUNK // 8) + 2 * t) + lax.shift_right_logical(_LANE, 3)
            ocol_base = lax.shift_left(_LANE & 7, 4)
            vals = [plsc.load_gather(bufs[b], [rows, sub16 + d])
                    for d in range(D)]
            for d in range(D):
                plsc.store_scatter(out_v, [orow, ocol_base + d], vals[d])

    prep_chunk(0)
    prep_chunk(1)
    fire(0, 0)
    fire(1, 1)

    def step(k, carry):
        for b in range(2):
            j = 2 * k + b
            pltpu.make_async_copy(
                tab_hbm.at[g_v.at[pl.ds(j * CHUNK, CHUNK)]], bufs[b],
                sems[b]).wait()
            extract(j, b)

            @pl.when(k < (NCHUNK // 2) - 1)
            def _():
                prep_chunk(j + 2)
                fire(j + 2, b)

            pltpu.async_copy(
                out_v.at[pl.ds(j * (CHUNK // 8), CHUNK // 8)],
                out_hbm.at[pl.ds(wid * OPW + j * (CHUNK // 8), CHUNK // 8)],
                osem)

        return carry

    lax.fori_loop(0, NCHUNK // 2, step, 0)

    for j in range(NCHUNK):
        pltpu.make_async_copy(
            out_v.at[pl.ds(j * (CHUNK // 8), CHUNK // 8)],
            out_hbm.at[pl.ds(wid * OPW + j * (CHUNK // 8), CHUNK // 8)],
            osem).wait()


_mesh = plsc.VectorSubcoreMesh(core_axis_name="c", subcore_axis_name="s")

_gather = pl.kernel(
    _body,
    out_type=jax.ShapeDtypeStruct((BF8, 128), jnp.float32),
    mesh=_mesh,
    scratch_types=[
        pltpu.VMEM((BPW,), jnp.int32),
        pltpu.VMEM((BPW,), jnp.int32),
        pltpu.VMEM((BPW,), jnp.int32),
        pltpu.VMEM((OPW, 128), jnp.float32),
        pltpu.VMEM((CHUNK, 128), jnp.float32),
        pltpu.VMEM((CHUNK, 128), jnp.float32),
        pltpu.SemaphoreType.DMA,
        pltpu.SemaphoreType.DMA,
        pltpu.SemaphoreType.DMA,
    ],
    compiler_params=pltpu.CompilerParams(needs_layout_passes=False),
)


@jax.jit
def kernel(indices, tables):
    idx_flat = indices.reshape(BF)
    tab128 = tables.reshape(FV8, 128)
    out = _gather(idx_flat, tab128)
    return out.reshape(B, F, D)


# speedup vs baseline: 9.9616x; 9.6795x over previous
import jax
import jax.numpy as jnp
from jax import lax
from jax.experimental import pallas as pl
from jax.experimental.pallas import tpu as pltpu
from jax.experimental.pallas import tpu_sc as plsc

B = 4096
F = 26
V = 100000
D = 16

_info = plsc.get_sparse_core_info()
NC = _info.num_cores
NS = _info.num_subcores
L = _info.num_lanes
NW = NC * NS
QPW = F * D // NW


def _body(idx_hbm, tab_hbm, out_hbm, row_v, idx_v, out_v, sem):
    wid = lax.axis_index("s") * NC + lax.axis_index("c")

    def pair(i, carry):
        q = wid * QPW + i
        f = q // D
        d = q - f * D
        pltpu.sync_copy(idx_hbm.at[f], idx_v)
        pltpu.sync_copy(tab_hbm.at[f, d], row_v)

        def grp(s, c):
            v16 = idx_v[pl.ds(s * L, L)]
            out_v[pl.ds(s * L, L)] = plsc.load_gather(row_v, [v16])
            return c

        lax.fori_loop(0, B // L, grp, 0)
        pltpu.sync_copy(out_v, out_hbm.at[f, d])
        return carry

    lax.fori_loop(0, QPW, pair, 0)


_mesh = plsc.VectorSubcoreMesh(core_axis_name="c", subcore_axis_name="s")

_gather = pl.kernel(
    _body,
    out_type=jax.ShapeDtypeStruct((F, D, B), jnp.float32),
    mesh=_mesh,
    scratch_types=[
        pltpu.VMEM((V,), jnp.float32),
        pltpu.VMEM((B,), jnp.int32),
        pltpu.VMEM((B,), jnp.float32),
        pltpu.SemaphoreType.DMA,
    ],
    compiler_params=pltpu.CompilerParams(needs_layout_passes=False),
)


@jax.jit
def kernel(indices, tables):
    idx_t = jnp.transpose(indices, (1, 0))
    tab_t = jnp.transpose(tables, (0, 2, 1))
    out_t = _gather(idx_t, tab_t)
    return jnp.transpose(out_t, (2, 0, 1))


# speedup vs baseline: 11.8348x; 1.1880x over previous
import jax
import jax.numpy as jnp
from jax import lax
from jax.experimental import pallas as pl
from jax.experimental.pallas import tpu as pltpu
from jax.experimental.pallas import tpu_sc as plsc

B = 4096
F = 26
V = 100000
D = 16

_info = plsc.get_sparse_core_info()
NC = _info.num_cores
NS = _info.num_subcores
L = _info.num_lanes
NW = NC * NS
QPW = F * D // NW
NG = B // L
UNROLL = 4

QSTART = (0, 25088, 50176, 75264)
QLEN = (25088, 25088, 25088, V - 75264)


def _body(idx_hbm, tab_hbm, out_hbm, idx_v, out_v, r0, r1, r2, r3,
          s0, s1, s2, s3, osem):
    rows = (r0, r1, r2, r3)
    sems = (s0, s1, s2, s3)
    wid = lax.axis_index("s") * NC + lax.axis_index("c")
    q0 = wid * QPW

    def fd(q):
        f = q // D
        return f, q - f * D

    f_0, d_0 = fd(q0)
    for h in range(4):
        pltpu.async_copy(
            tab_hbm.at[f_0, d_0, pl.ds(QSTART[h], QLEN[h])], rows[h], sems[h])

    def pair(i, f_prev):
        q = q0 + i
        f, d = fd(q)
        f1, d1 = fd(q + 1)

        @pl.when(i > 0)
        def _():
            pltpu.make_async_copy(out_v, out_hbm.at[f, d], osem).wait()

        @pl.when(f != f_prev)
        def _():
            pltpu.sync_copy(idx_hbm.at[f], idx_v)

        for h in range(4):
            pltpu.make_async_copy(
                tab_hbm.at[f, d, pl.ds(QSTART[h], QLEN[h])], rows[h],
                sems[h]).wait()

            def grp(s, c, h=h):
                for m in range(UNROLL):
                    o = s * (L * UNROLL) + m * L
                    v16 = idx_v[pl.ds(o, L)]
                    vl = jnp.minimum(
                        jnp.maximum(v16 - QSTART[h], 0), QLEN[h] - 1)
                    vals = plsc.load_gather(rows[h], [vl])
                    if h == 0:
                        out_v[pl.ds(o, L)] = vals
                    else:
                        keep = v16 >= QSTART[h]
                        out_v[pl.ds(o, L)] = jnp.where(
                            keep, vals, out_v[pl.ds(o, L)])
                return c

            lax.fori_loop(0, NG // UNROLL, grp, 0)

            @pl.when(i < QPW - 1)
            def _(h=h):
                pltpu.async_copy(
                    tab_hbm.at[f1, d1, pl.ds(QSTART[h], QLEN[h])], rows[h],
                    sems[h])

        pltpu.async_copy(out_v, out_hbm.at[f, d], osem)
        return f

    lax.fori_loop(0, QPW, pair, jnp.int32(-1))
    f_l, d_l = fd(q0 + QPW - 1)
    pltpu.make_async_copy(out_v, out_hbm.at[f_l, d_l], osem).wait()


_mesh = plsc.VectorSubcoreMesh(core_axis_name="c", subcore_axis_name="s")

_gather = pl.kernel(
    _body,
    out_type=jax.ShapeDtypeStruct((F, D, B), jnp.float32),
    mesh=_mesh,
    scratch_types=[
        pltpu.VMEM((B,), jnp.int32),
        pltpu.VMEM((B,), jnp.float32),
        pltpu.VMEM((QLEN[0],), jnp.float32),
        pltpu.VMEM((QLEN[1],), jnp.float32),
        pltpu.VMEM((QLEN[2],), jnp.float32),
        pltpu.VMEM((QLEN[3],), jnp.float32),
        pltpu.SemaphoreType.DMA,
        pltpu.SemaphoreType.DMA,
        pltpu.SemaphoreType.DMA,
        pltpu.SemaphoreType.DMA,
        pltpu.SemaphoreType.DMA,
    ],
    compiler_params=pltpu.CompilerParams(needs_layout_passes=False),
)


@jax.jit
def kernel(indices, tables):
    idx_t = jnp.transpose(indices, (1, 0))
    tab_t = jnp.transpose(tables, (0, 2, 1))
    out_t = _gather(idx_t, tab_t)
    return jnp.transpose(out_t, (2, 0, 1))


# speedup vs baseline: 13.1140x; 1.1081x over previous
import jax
import jax.numpy as jnp
from jax import lax
from jax.experimental import pallas as pl
from jax.experimental.pallas import tpu as pltpu
from jax.experimental.pallas import tpu_sc as plsc

B = 4096
F = 26
V = 100000
D = 16

_info = plsc.get_sparse_core_info()
NC = _info.num_cores
NS = _info.num_subcores
L = _info.num_lanes
NW = NC * NS
QPW = F * D // NW
NG = B // L
UNROLL = 4

SPLIT = 50176
HLEN = (SPLIT, V - SPLIT)


def _body(idx_hbm, tab_hbm, out_hbm, idx_v, out_v, r0, r1, s0, s1, osem):
    rows = (r0, r1)
    sems = (s0, s1)
    wid = lax.axis_index("s") * NC + lax.axis_index("c")
    q0 = wid * QPW

    def fd(q):
        f = q // D
        return f, q - f * D

    def stage(f, d, h):
        return pltpu.async_copy(
            tab_hbm.at[f, d, pl.ds(h * SPLIT, HLEN[h])], rows[h], sems[h])

    def stage_wait(f, d, h):
        pltpu.make_async_copy(
            tab_hbm.at[f, d, pl.ds(h * SPLIT, HLEN[h])], rows[h],
            sems[h]).wait()

    f_0, d_0 = fd(q0)
    stage(f_0, d_0, 0)
    stage(f_0, d_0, 1)

    def pair(i, f_prev):
        q = q0 + i
        f, d = fd(q)
        f1, d1 = fd(q + 1)

        @pl.when(i > 0)
        def _():
            pltpu.make_async_copy(out_v, out_hbm.at[f, d], osem).wait()

        @pl.when(f != f_prev)
        def _():
            pltpu.sync_copy(idx_hbm.at[f], idx_v)

        stage_wait(f, d, 0)

        def grp0(s, c):
            for m in range(UNROLL):
                o = s * (L * UNROLL) + m * L
                v16 = idx_v[pl.ds(o, L)]
                vl = jnp.minimum(v16, SPLIT - 1)
                out_v[pl.ds(o, L)] = plsc.load_gather(r0, [vl])
            return c

        lax.fori_loop(0, NG // UNROLL, grp0, 0)

        @pl.when(i < QPW - 1)
        def _():
            stage(f1, d1, 0)

        stage_wait(f, d, 1)

        def grp1(s, c):
            for m in range(UNROLL):
                o = s * (L * UNROLL) + m * L
                v16 = idx_v[pl.ds(o, L)]
                vl = jnp.maximum(v16 - SPLIT, 0)
                vals = plsc.load_gather(r1, [vl])
                out_v[pl.ds(o, L)] = jnp.where(
                    v16 >= SPLIT, vals, out_v[pl.ds(o, L)])
            return c

        lax.fori_loop(0, NG // UNROLL, grp1, 0)

        @pl.when(i < QPW - 1)
        def _():
            stage(f1, d1, 1)

        pltpu.async_copy(out_v, out_hbm.at[f, d], osem)
        return f

    lax.fori_loop(0, QPW, pair, jnp.int32(-1))
    f_l, d_l = fd(q0 + QPW - 1)
    pltpu.make_async_copy(out_v, out_hbm.at[f_l, d_l], osem).wait()


_mesh = plsc.VectorSubcoreMesh(core_axis_name="c", subcore_axis_name="s")

_gather = pl.kernel(
    _body,
    out_type=jax.ShapeDtypeStruct((F, D, B), jnp.float32),
    mesh=_mesh,
    scratch_types=[
        pltpu.VMEM((B,), jnp.int32),
        pltpu.VMEM((B,), jnp.float32),
        pltpu.VMEM((HLEN[0],), jnp.float32),
        pltpu.VMEM((HLEN[1],), jnp.float32),
        pltpu.SemaphoreType.DMA,
        pltpu.SemaphoreType.DMA,
        pltpu.SemaphoreType.DMA,
    ],
    compiler_params=pltpu.CompilerParams(needs_layout_passes=False),
)


@jax.jit
def kernel(indices, tables):
    idx_t = jnp.transpose(indices, (1, 0))
    tab_t = jnp.transpose(tables, (0, 2, 1))
    out_t = _gather(idx_t, tab_t)
    return jnp.transpose(out_t, (2, 0, 1))
